# baseline (speedup 1.0000x reference)
import functools

import jax
import jax.numpy as jnp
from jax import lax
from jax.experimental import pallas as pl
from jax.experimental.pallas import tpu as pltpu

_GRID = 2


def _body(a_ref, b_ref, g_ref, w1_ref, b1_ref, w2_ref, b2_ref,
          o_ref, acc_e, acc_n, *, grid, inv_e, inv_n, d_edge, d_global):
    i = pl.program_id(0)
    blk = a_ref.shape[1]
    ea = a_ref[...].reshape(d_edge, blk // 128, 128).sum(axis=1)
    na = jnp.sum(b_ref[...], axis=0, keepdims=True)

    @pl.when(i == 0)
    def _init():
        acc_e[...] = ea
        acc_n[0:1, :] = na

    @pl.when(i > 0)
    def _acc():
        acc_e[...] = acc_e[...] + ea
        acc_n[0:1, :] = acc_n[0:1, :] + na

    @pl.when(i == grid - 1)
    def _finish():
        s16 = jnp.sum(acc_e[...], axis=1, keepdims=True) * inv_e
        nmean = acc_n[0:1, :] * inv_n
        wg = w1_ref[:d_global, :]
        we = w1_ref[d_global:d_global + d_edge, :]
        wn = w1_ref[d_global + d_edge:, :]
        e_pre = lax.dot_general(s16, we, (((0,), (0,)), ((), ())))
        pre = (g_ref[...] @ wg + e_pre + nmean @ wn + b1_ref[...][None, :])
        h = jnp.maximum(pre, 0.0)
        o_ref[...] = h @ w2_ref[...] + b2_ref[...][None, :]


def kernel(node_attr, edge_index, edge_attr, global_attr, W1, b1, W2, b2):
    del edge_index
    n_edges, d_edge = edge_attr.shape
    n_nodes, d_feat = node_attr.shape
    d_global = global_attr.shape[1]
    in_features, latent = W1.shape
    out_features = W2.shape[1]

    et = edge_attr.T

    grid = _GRID
    blk_a = n_edges // grid
    blk_b = n_nodes // grid

    body = functools.partial(_body, grid=grid, inv_e=1.0 / n_edges,
                             inv_n=1.0 / n_nodes, d_edge=d_edge,
                             d_global=d_global)
    out = pl.pallas_call(
        body,
        grid=(grid,),
        in_specs=[
            pl.BlockSpec((d_edge, blk_a), lambda i: (0, i)),
            pl.BlockSpec((blk_b, d_feat), lambda i: (i, 0)),
            pl.BlockSpec((1, d_global), lambda i: (0, 0)),
            pl.BlockSpec((in_features, latent), lambda i: (0, 0)),
            pl.BlockSpec((latent,), lambda i: (0,)),
            pl.BlockSpec((latent, out_features), lambda i: (0, 0)),
            pl.BlockSpec((out_features,), lambda i: (0,)),
        ],
        out_specs=pl.BlockSpec((1, out_features), lambda i: (0, 0)),
        out_shape=jax.ShapeDtypeStruct((1, out_features), jnp.float32),
        scratch_shapes=[pltpu.VMEM((16, 128), jnp.float32),
                        pltpu.VMEM((8, 128), jnp.float32)],
    )(et, node_attr, global_attr, W1, b1, W2, b2)
    return out


# speedup vs baseline: 1.0008x; 1.0008x over previous
import functools

import jax
import jax.numpy as jnp
from jax import lax
from jax.experimental import pallas as pl
from jax.experimental.pallas import tpu as pltpu

_GRID = 5


def _body(a_ref, b_ref, g_ref, w1_ref, b1_ref, w2_ref, b2_ref,
          o_ref, acc_e, acc_n, *, grid, inv_e, inv_n, d_edge, d_global):
    i = pl.program_id(0)
    blk = a_ref.shape[1]
    ea = a_ref[...].reshape(d_edge, blk // 128, 128).sum(axis=1)
    na = jnp.sum(b_ref[...], axis=0, keepdims=True)

    @pl.when(i == 0)
    def _init():
        acc_e[...] = ea
        acc_n[0:1, :] = na

    @pl.when(i > 0)
    def _acc():
        acc_e[...] = acc_e[...] + ea
        acc_n[0:1, :] = acc_n[0:1, :] + na

    @pl.when(i == grid - 1)
    def _finish():
        s16 = jnp.sum(acc_e[...], axis=1, keepdims=True) * inv_e
        nmean = acc_n[0:1, :] * inv_n
        wg = w1_ref[:d_global, :]
        we = w1_ref[d_global:d_global + d_edge, :]
        wn = w1_ref[d_global + d_edge:, :]
        e_pre = lax.dot_general(s16, we, (((0,), (0,)), ((), ())))
        pre = (g_ref[...] @ wg + e_pre + nmean @ wn + b1_ref[...][None, :])
        h = jnp.maximum(pre, 0.0)
        o_ref[...] = h @ w2_ref[...] + b2_ref[...][None, :]


def kernel(node_attr, edge_index, edge_attr, global_attr, W1, b1, W2, b2):
    del edge_index
    n_edges, d_edge = edge_attr.shape
    n_nodes, d_feat = node_attr.shape
    d_global = global_attr.shape[1]
    in_features, latent = W1.shape
    out_features = W2.shape[1]

    et = edge_attr.T

    grid = _GRID
    blk_a = n_edges // grid
    blk_b = n_nodes // grid

    body = functools.partial(_body, grid=grid, inv_e=1.0 / n_edges,
                             inv_n=1.0 / n_nodes, d_edge=d_edge,
                             d_global=d_global)
    out = pl.pallas_call(
        body,
        grid=(grid,),
        in_specs=[
            pl.BlockSpec((d_edge, blk_a), lambda i: (0, i)),
            pl.BlockSpec((blk_b, d_feat), lambda i: (i, 0)),
            pl.BlockSpec((1, d_global), lambda i: (0, 0)),
            pl.BlockSpec((in_features, latent), lambda i: (0, 0)),
            pl.BlockSpec((latent,), lambda i: (0,)),
            pl.BlockSpec((latent, out_features), lambda i: (0, 0)),
            pl.BlockSpec((out_features,), lambda i: (0,)),
        ],
        out_specs=pl.BlockSpec((1, out_features), lambda i: (0, 0)),
        out_shape=jax.ShapeDtypeStruct((1, out_features), jnp.float32),
        scratch_shapes=[pltpu.VMEM((16, 128), jnp.float32),
                        pltpu.VMEM((8, 128), jnp.float32)],
    )(et, node_attr, global_attr, W1, b1, W2, b2)
    return out
